# baseline (speedup 1.0000x reference)
import functools

import jax
import jax.numpy as jnp
from jax import lax
from jax.experimental import pallas as pl
from jax.experimental.pallas import tpu as pltpu
from jax.experimental.pallas import tpu_sc as plsc

_D = 128
_NC = 2
_NS = 16
_NW = _NC * _NS
_G = 2
_NB = 3


def _emb_body(T, iblk, x_hbm, w_hbm, out_hbm, idx_v, buf_v,
              g0, g1, g2, w0, w1, w2):
    wid = lax.axis_index("s") * _NC + lax.axis_index("c")
    i0 = wid * iblk
    nunits = T // _G

    pltpu.sync_copy(x_hbm.at[:, pl.ds(i0, iblk)], idx_v)

    gsems = (g0, g1, g2)
    wsems = (w0, w1, w2)

    def gathers(u, b):
        return [
            pltpu.make_async_copy(
                w_hbm.at[idx_v.at[u * _G + j]], buf_v.at[b, j], gsems[b])
            for j in range(_G)
        ]

    def fire(u, b):
        for c in gathers(u, b):
            c.start()

    def drain(u, b):
        for c in gathers(u, b):
            c.wait()

    def wdesc(u, b):
        return pltpu.make_async_copy(
            buf_v.at[b], out_hbm.at[pl.ds(u * _G, _G), pl.ds(i0, iblk)],
            wsems[b])

    fire(0, 0)
    fire(1, 1)

    drain(0, 0)
    wdesc(0, 0).start()
    fire(2, 2)

    def body(i, carry):
        for jj in range(_NB):
            u = 3 * i + 1 + jj
            b = (1 + jj) % _NB
            bn = (b + 2) % _NB
            drain(u, b)
            wdesc(u, b).start()
            wdesc(0, bn).wait()
            fire(u + 2, bn)
        return carry

    lax.fori_loop(0, 7, body, 0)

    u = nunits - 3
    drain(u, 1)
    wdesc(u, 1).start()
    wdesc(0, 0).wait()
    fire(u + 2, 0)
    drain(u + 1, 2)
    wdesc(u + 1, 2).start()
    drain(u + 2, 0)
    wdesc(u + 2, 0).start()
    wdesc(0, 1).wait()
    wdesc(0, 2).wait()
    wdesc(0, 0).wait()


def kernel(x, weight):
    S, T = x.shape
    iblk = S // _NW
    xt = jnp.transpose(x.astype(jnp.int32))

    mesh = plsc.VectorSubcoreMesh(core_axis_name="c", subcore_axis_name="s")
    k = pl.kernel(
        functools.partial(_emb_body, T, iblk),
        out_type=jax.ShapeDtypeStruct((T, S, _D), jnp.float32),
        mesh=mesh,
        scratch_types=[
            pltpu.VMEM((T, iblk), jnp.int32),
            pltpu.VMEM((_NB, _G, iblk, _D), jnp.float32),
            pltpu.SemaphoreType.DMA,
            pltpu.SemaphoreType.DMA,
            pltpu.SemaphoreType.DMA,
            pltpu.SemaphoreType.DMA,
            pltpu.SemaphoreType.DMA,
            pltpu.SemaphoreType.DMA,
        ],
    )
    out_t = k(xt, weight)
    return jnp.transpose(out_t, (1, 0, 2))


# speedup vs baseline: 1.0119x; 1.0119x over previous
import functools

import jax
import jax.numpy as jnp
from jax import lax
from jax.experimental import pallas as pl
from jax.experimental.pallas import tpu as pltpu
from jax.experimental.pallas import tpu_sc as plsc

_D = 128
_NC = 2
_NS = 16
_NW = _NC * _NS
_G = 3


def _emb_body(T, iblk, x_hbm, w_hbm, out_hbm, idx_v, buf_v, g0, g1):
    wid = lax.axis_index("s") * _NC + lax.axis_index("c")
    i0 = wid * iblk
    nmain = T // _G
    tail = T - nmain * _G

    pltpu.sync_copy(x_hbm.at[:, pl.ds(i0, iblk)], idx_v)

    sems = (g0, g1)

    def gathers(u, b, n=_G):
        return [
            pltpu.make_async_copy(
                w_hbm.at[idx_v.at[u * _G + j]], buf_v.at[b, j], sems[b])
            for j in range(n)
        ]

    def fire(u, b, n=_G):
        for c in gathers(u, b, n):
            c.start()

    def drain(u, b, n=_G):
        for c in gathers(u, b, n):
            c.wait()

    def write(u, b, n=_G):
        pltpu.sync_copy(buf_v.at[b, pl.ds(0, n)],
                        out_hbm.at[pl.ds(u * _G, n), pl.ds(i0, iblk)])

    fire(0, 0)
    fire(1, 1)

    def body(i, carry):
        for b in range(2):
            u = 2 * i + b
            drain(u, b)
            write(u, b)
            fire(u + 2, b)
        return carry

    lax.fori_loop(0, nmain // 2 - 1, body, 0)

    u = nmain - 2
    drain(u, 0)
    write(u, 0)
    fire(nmain, 0, n=tail)
    drain(u + 1, 1)
    write(u + 1, 1)
    drain(nmain, 0, n=tail)
    write(nmain, 0, n=tail)


def kernel(x, weight):
    S, T = x.shape
    iblk = S // _NW
    xt = jnp.transpose(x.astype(jnp.int32))

    mesh = plsc.VectorSubcoreMesh(core_axis_name="c", subcore_axis_name="s")
    k = pl.kernel(
        functools.partial(_emb_body, T, iblk),
        out_type=jax.ShapeDtypeStruct((T, S, _D), jnp.float32),
        mesh=mesh,
        scratch_types=[
            pltpu.VMEM((T, iblk), jnp.int32),
            pltpu.VMEM((2, _G, iblk, _D), jnp.float32),
            pltpu.SemaphoreType.DMA,
            pltpu.SemaphoreType.DMA,
        ],
    )
    out_t = k(xt, weight)
    return jnp.transpose(out_t, (1, 0, 2))
